# Initial kernel scaffold; baseline (speedup 1.0000x reference)
import jax
import jax.numpy as jnp
from jax.experimental import pallas as pl


def kernel(x, weight):
    raise NotImplementedError("write your pallas kernel here")


# speedup vs baseline: 85.3112x; 85.3112x over previous
import functools

import jax
import jax.numpy as jnp
from jax import lax
from jax.experimental import pallas as pl
from jax.experimental.pallas import tpu as pltpu
from jax.experimental.pallas import tpu_sc as plsc

B = 16384
LBAG = 200
V = 100
D = 32

NC = 2
NS = 16
NW = NC * NS
BW = B // NW
CB = 256
NCHUNK = BW // CB

_mesh = plsc.VectorSubcoreMesh(core_axis_name="c", subcore_axis_name="s")


@functools.partial(
    pl.kernel,
    mesh=_mesh,
    out_type=jax.ShapeDtypeStruct((B * V,), jnp.float32),
    compiler_params=pltpu.CompilerParams(needs_layout_passes=False),
    scratch_types=[
        pltpu.VMEM((CB * LBAG + 16,), jnp.int32),
        pltpu.VMEM((CB * V,), jnp.float32),
    ],
)
def _hist(x_hbm, counts_hbm, xv, cv):
    wid = lax.axis_index("s") * NC + lax.axis_index("c")
    iot = lax.iota(jnp.int32, 16)
    mask8 = iot < 8
    ones = jnp.ones((16,), jnp.float32)

    for chunk in range(NCHUNK):
        bag0 = wid * BW + chunk * CB
        pltpu.sync_copy(
            x_hbm.at[pl.ds(bag0 * LBAG, CB * LBAG)],
            xv.at[pl.ds(0, CB * LBAG)],
        )

        def zbody(j, carry):
            cv[pl.ds(j * 16, 16)] = jnp.zeros((16,), jnp.float32)
            return carry

        lax.fori_loop(0, CB * V // 16, zbody, 0)

        def bag_body(bag, carry):
            xoff = bag * LBAG
            dbase = bag * V
            for i in range(12):
                xvec = xv[pl.ds(xoff + i * 16, 16)]
                plsc.addupdate_scatter(cv, [xvec + dbase], ones)
            xvec = xv[pl.ds(xoff + 192, 16)]
            plsc.addupdate_scatter(cv, [xvec + dbase], ones, mask=mask8)
            return carry

        lax.fori_loop(0, CB, bag_body, 0)

        pltpu.sync_copy(cv, counts_hbm.at[pl.ds(bag0 * V, CB * V)])


def _matmul(counts, weight):
    BM = 1024

    def body(c_ref, w_ref, o_ref):
        o_ref[...] = jnp.dot(
            c_ref[...], w_ref[...], preferred_element_type=jnp.float32
        )

    return pl.pallas_call(
        body,
        grid=(B // BM,),
        in_specs=[
            pl.BlockSpec((BM, V), lambda i: (i, 0)),
            pl.BlockSpec((V, D), lambda i: (0, 0)),
        ],
        out_specs=pl.BlockSpec((BM, D), lambda i: (i, 0)),
        out_shape=jax.ShapeDtypeStruct((B, D), jnp.float32),
    )(counts, weight)


def kernel(x, weight):
    x_flat = x.astype(jnp.int32).reshape(-1)
    counts = _hist(x_flat).reshape(B, V)
    return _matmul(counts, weight)
